# baseline (speedup 1.0000x reference)
import functools

import jax
import jax.numpy as jnp
from jax import lax
from jax.experimental import pallas as pl
from jax.experimental.pallas import tpu as pltpu
from jax.experimental.pallas import tpu_sc as plsc

B = 16384
ED = 32
HL = 64
G = 20
NC, NS = 2, 16
NW = NC * NS
BPW = B // NW
CHUNK = 128
NCH = BPW // CHUNK
NG = BPW // 16


@functools.cache
def _sc_gather_fn():
    mesh = plsc.VectorSubcoreMesh(core_axis_name="c", subcore_axis_name="s",
                                  num_cores=NC, num_subcores=NS)

    @functools.partial(
        pl.kernel,
        out_type=(
            jax.ShapeDtypeStruct((ED, B // CHUNK, CHUNK), jnp.float32),
            jax.ShapeDtypeStruct((B,), jnp.float32),
            jax.ShapeDtypeStruct((B,), jnp.float32),
        ),
        mesh=mesh,
        compiler_params=pltpu.CompilerParams(use_tc_tiling_on_sc=False,
                                             needs_layout_passes=False),
        scratch_types=(
            pltpu.VMEM((NCH, CHUNK), jnp.int32),
            pltpu.VMEM((NCH, CHUNK), jnp.int32),
            pltpu.VMEM((BPW, ED), jnp.float32),
            pltpu.VMEM((ED, NCH, CHUNK), jnp.float32),
            pltpu.VMEM((BPW,), jnp.float32),
            pltpu.VMEM((BPW,), jnp.float32),
            pltpu.SemaphoreType.DMA,
            pltpu.SemaphoreType.DMA,
        ),
    )
    def _sc_gather(uidx_hbm, midx_hbm, memb_hbm, ubias_hbm, mbias_hbm,
                   mrowsT_out, ub_out, mb_out,
                   uidx_v, midx_v, buf, mT_v, ub_v, mb_v, sem_g, sem_b):
        wid = lax.axis_index("s") * NC + lax.axis_index("c")
        base = wid * BPW
        pltpu.sync_copy(uidx_hbm.at[wid], uidx_v)
        pltpu.sync_copy(midx_hbm.at[wid], midx_v)
        bias_copies = []
        for j in range(NCH):
            sl = pl.ds(j * CHUNK, CHUNK)
            bias_copies.append(pltpu.async_copy(ubias_hbm.at[uidx_v.at[j]], ub_v.at[sl], sem_b))
            bias_copies.append(pltpu.async_copy(mbias_hbm.at[midx_v.at[j]], mb_v.at[sl], sem_b))
        g_copies = []
        for j in range(NCH):
            sl = pl.ds(j * CHUNK, CHUNK)
            g_copies.append(pltpu.async_copy(memb_hbm.at[midx_v.at[j]], buf.at[sl], sem_g))
        for c in g_copies:
            c.wait()

        iota16 = lax.iota(jnp.int32, 16)
        cvecs = [iota16 * 0 + c for c in range(ED)]

        def extract(g, carry):
            j = g >> 3
            gg = g & 7
            rowv = g * 16 + iota16
            for c in range(ED):
                v = plsc.load_gather(buf, [rowv, cvecs[c]])
                mT_v[c, j, pl.ds(gg * 16, 16)] = v
            return carry

        lax.fori_loop(0, NG, extract, 0, unroll=4)

        for c in bias_copies:
            c.wait()
        pltpu.sync_copy(mT_v, mrowsT_out.at[:, pl.ds(wid * NCH, NCH), :])
        pltpu.sync_copy(ub_v, ub_out.at[pl.ds(base, BPW)])
        pltpu.sync_copy(mb_v, mb_out.at[pl.ds(base, BPW)])

    return _sc_gather


BLK = 2048


def _tc_body(uT_ref, mT_ref, gT_ref, ub_ref, mb_ref, gW_ref, gb_ref,
             w1_ref, b1_ref, w2_ref, c2_ref, out_ref):
    cdims = (((1,), (0,)), ((), ()))
    uT = uT_ref[...]
    mT = mT_ref[...].reshape(ED, BLK)
    w1 = w1_ref[...]
    geT = lax.dot_general(gW_ref[...], gT_ref[...], cdims,
                          preferred_element_type=jnp.float32) + gb_ref[...]
    pre = lax.dot_general(w1[:, :ED], uT, cdims, preferred_element_type=jnp.float32)
    pre = pre + lax.dot_general(w1[:, ED:2 * ED], mT, cdims,
                                preferred_element_type=jnp.float32)
    pre = pre + lax.dot_general(w1[:, 2 * ED:], geT, cdims,
                                preferred_element_type=jnp.float32)
    h = jnp.maximum(pre + b1_ref[...], 0.0)
    mlpT = lax.dot_general(w2_ref[...], h, cdims, preferred_element_type=jnp.float32)
    dp = jnp.sum(uT * mT, axis=0)
    out_ref[...] = dp + mlpT[0, :] + ub_ref[...] + mb_ref[...] + c2_ref[0, 0]


def _tc_forward(uT, mT, gT, ub, mb, gW, gb2, W1, b12, W2, c2):
    return pl.pallas_call(
        _tc_body,
        grid=(B // BLK,),
        in_specs=[
            pl.BlockSpec((ED, BLK), lambda i: (0, i)),
            pl.BlockSpec((ED, BLK // CHUNK, CHUNK), lambda i: (0, i, 0)),
            pl.BlockSpec((G, BLK), lambda i: (0, i)),
            pl.BlockSpec((BLK,), lambda i: (i,)),
            pl.BlockSpec((BLK,), lambda i: (i,)),
            pl.BlockSpec((ED, G), lambda i: (0, 0)),
            pl.BlockSpec((ED, 1), lambda i: (0, 0)),
            pl.BlockSpec((HL, 3 * ED), lambda i: (0, 0)),
            pl.BlockSpec((HL, 1), lambda i: (0, 0)),
            pl.BlockSpec((1, HL), lambda i: (0, 0)),
            pl.BlockSpec((1, 1), lambda i: (0, 0)),
        ],
        out_specs=pl.BlockSpec((BLK,), lambda i: (i,)),
        out_shape=jax.ShapeDtypeStruct((B,), jnp.float32),
    )(uT, mT, gT, ub, mb, gW, gb2, W1, b12, W2, c2)


def kernel(userIndices, movieIndices, genreIndeces, userEmb, movieEmb,
           userBiasT, movieBiasT, bias, gW, gb, W1, b1, W2, b2):
    uidx = userIndices.astype(jnp.int32).reshape(NW, NCH, CHUNK)
    midx = movieIndices.astype(jnp.int32).reshape(NW, NCH, CHUNK)
    urowsT = jnp.take(userEmb, userIndices, axis=0, mode="clip").T
    mT, ub, mb = _sc_gather_fn()(
        uidx, midx, movieEmb,
        userBiasT.reshape(-1), movieBiasT.reshape(-1))
    c2 = (bias + b2).reshape(1, 1)
    return _tc_forward(urowsT, mT, genreIndeces.T, ub, mb, gW,
                       gb.reshape(ED, 1), W1,
                       b1.reshape(HL, 1), W2, c2)
